# Initial kernel scaffold; baseline (speedup 1.0000x reference)
import jax
import jax.numpy as jnp
from jax.experimental import pallas as pl


def kernel(encoder_hidden_states, label, encoder_attention_mask, embedding_table):
    raise NotImplementedError("write your pallas kernel here")


# speedup vs baseline: 1.2318x; 1.2318x over previous
import functools

import jax
import jax.numpy as jnp
from jax import lax
from jax.experimental import pallas as pl
from jax.experimental.pallas import tpu as pltpu
from jax.experimental.pallas import tpu_sc as plsc

VOCAB = 32128
D_MODEL = 1024
BATCH = 4
SEQ = 2048
N_TOK = BATCH * SEQ
NC, NS = 2, 16
NW = NC * NS
ROWS_PER_W = N_TOK // NW
CHUNK = 32
NCHUNK = ROWS_PER_W // CHUNK
PAD = 9
LANES = 16

DECODER_START_TOKEN_ID = 0
PAD_TOKEN_ID = 0


def _emb_body(label_hbm, table_hbm, out_hbm, mask_hbm,
              lbl_v, ids_v, buf0, buf1, ones_v, sg0, sg1, sw0, sw1):
    wid = lax.axis_index("s") * NC + lax.axis_index("c")
    base = pl.multiple_of(wid * ROWS_PER_W, ROWS_PER_W)

    pltpu.sync_copy(label_hbm.at[pl.ds(base, ROWS_PER_W + PAD - 1)], lbl_v)

    lane = lax.iota(jnp.int32, LANES)
    ones16 = jnp.full((LANES,), 1.0, jnp.float32)
    for j in range(ROWS_PER_W // LANES):
        n_vec = base + j * LANES + lane
        is_t0 = (n_vec & (SEQ - 1)) == 0
        ids = lbl_v[pl.ds(PAD - 1 + j * LANES, LANES)]
        ids = jnp.where(ids == -100, PAD_TOKEN_ID, ids)
        ids = jnp.where(is_t0, DECODER_START_TOKEN_ID, ids)
        k, o = divmod(j * LANES, CHUNK)
        ids_v[k, pl.ds(o, LANES)] = ids
        ones_v[pl.ds(j * LANES, LANES)] = ones16
    pltpu.sync_copy(ones_v, mask_hbm.at[pl.ds(base, ROWS_PER_W)])

    bufs, sgs, sws = [buf0, buf1], [sg0, sg1], [sw0, sw1]

    def start_gather(k):
        return pltpu.async_copy(table_hbm.at[ids_v.at[k]], bufs[k % 2], sgs[k % 2])

    writes = [None] * NCHUNK
    g = start_gather(0)
    for k in range(NCHUNK):
        g.wait()
        writes[k] = pltpu.async_copy(
            bufs[k % 2], out_hbm.at[pl.ds(base + k * CHUNK, CHUNK)], sws[k % 2])
        if k + 1 < NCHUNK:
            if k >= 1:
                writes[k - 1].wait()
            g = start_gather(k + 1)
    writes[NCHUNK - 2].wait()
    writes[NCHUNK - 1].wait()


@functools.partial(
    pl.kernel,
    out_type=(jax.ShapeDtypeStruct((N_TOK, D_MODEL), jnp.float32),
              jax.ShapeDtypeStruct((N_TOK,), jnp.float32)),
    mesh=plsc.VectorSubcoreMesh(core_axis_name="c", subcore_axis_name="s",
                                num_cores=NC, num_subcores=NS),
    scratch_types=[
        pltpu.VMEM((ROWS_PER_W + PAD - 1,), jnp.int32),
        pltpu.VMEM((NCHUNK, CHUNK), jnp.int32),
        pltpu.VMEM((CHUNK, D_MODEL), jnp.float32),
        pltpu.VMEM((CHUNK, D_MODEL), jnp.float32),
        pltpu.VMEM((ROWS_PER_W,), jnp.float32),
        pltpu.SemaphoreType.DMA,
        pltpu.SemaphoreType.DMA,
        pltpu.SemaphoreType.DMA,
        pltpu.SemaphoreType.DMA,
    ],
)
def _emb_lookup(label_hbm, table_hbm, out_hbm, mask_hbm,
                lbl_v, ids_v, buf0, buf1, ones_v, sg0, sg1, sw0, sw1):
    _emb_body(label_hbm, table_hbm, out_hbm, mask_hbm,
              lbl_v, ids_v, buf0, buf1, ones_v, sg0, sg1, sw0, sw1)


def kernel(encoder_hidden_states, label, encoder_attention_mask, embedding_table):
    label_padded = jnp.concatenate(
        [jnp.zeros((PAD,), jnp.int32), label.reshape(N_TOK)])
    out, mask = _emb_lookup(label_padded, embedding_table)
    return (encoder_hidden_states, encoder_attention_mask,
            out.reshape(BATCH, SEQ, D_MODEL), mask.reshape(BATCH, SEQ))


# speedup vs baseline: 1.2571x; 1.0205x over previous
import functools

import jax
import jax.numpy as jnp
from jax import lax
from jax.experimental import pallas as pl
from jax.experimental.pallas import tpu as pltpu
from jax.experimental.pallas import tpu_sc as plsc

VOCAB = 32128
D_MODEL = 1024
BATCH = 4
SEQ = 2048
N_TOK = BATCH * SEQ
NC, NS = 2, 16
NW = NC * NS
ROWS_PER_W = N_TOK // NW
CHUNK = 32
NCHUNK = ROWS_PER_W // CHUNK
PAD = 9
LANES = 16

DECODER_START_TOKEN_ID = 0
PAD_TOKEN_ID = 0


def _emb_body(label_hbm, table_hbm, out_hbm, mask_hbm,
              lbl_v, ids_v, buf0, buf1, ones_v, sg0, sg1, sw0, sw1):
    wid = lax.axis_index("s") * NC + lax.axis_index("c")
    base = pl.multiple_of(wid * ROWS_PER_W, ROWS_PER_W)
    b = wid // (SEQ // ROWS_PER_W)
    t_base = pl.multiple_of((wid % (SEQ // ROWS_PER_W)) * ROWS_PER_W,
                            ROWS_PER_W)

    pltpu.sync_copy(label_hbm.at[pl.ds(base, ROWS_PER_W + PAD - 1)], lbl_v)

    lane = lax.iota(jnp.int32, LANES)
    ones16 = jnp.full((LANES,), 1.0, jnp.float32)
    for j in range(ROWS_PER_W // LANES):
        n_vec = base + j * LANES + lane
        is_t0 = (n_vec & (SEQ - 1)) == 0
        ids = lbl_v[pl.ds(PAD - 1 + j * LANES, LANES)]
        ids = jnp.where(ids == -100, PAD_TOKEN_ID, ids)
        ids = jnp.where(is_t0, DECODER_START_TOKEN_ID, ids)
        k, o = divmod(j * LANES, CHUNK)
        ids_v[k, pl.ds(o, LANES)] = ids
        ones_v[pl.ds(j * LANES, LANES)] = ones16
    pltpu.sync_copy(ones_v, mask_hbm.at[b, pl.ds(t_base, ROWS_PER_W)])

    bufs, sgs, sws = [buf0, buf1], [sg0, sg1], [sw0, sw1]

    def start_gather(k):
        return pltpu.async_copy(table_hbm.at[ids_v.at[k]], bufs[k % 2], sgs[k % 2])

    writes = [None] * NCHUNK
    g = start_gather(0)
    for k in range(NCHUNK):
        g.wait()
        writes[k] = pltpu.async_copy(
            bufs[k % 2], out_hbm.at[b, pl.ds(t_base + k * CHUNK, CHUNK)],
            sws[k % 2])
        if k + 1 < NCHUNK:
            if k >= 1:
                writes[k - 1].wait()
            g = start_gather(k + 1)
    writes[NCHUNK - 2].wait()
    writes[NCHUNK - 1].wait()


@functools.partial(
    pl.kernel,
    out_type=(jax.ShapeDtypeStruct((BATCH, SEQ, D_MODEL), jnp.float32),
              jax.ShapeDtypeStruct((BATCH, SEQ), jnp.float32)),
    mesh=plsc.VectorSubcoreMesh(core_axis_name="c", subcore_axis_name="s",
                                num_cores=NC, num_subcores=NS),
    scratch_types=[
        pltpu.VMEM((ROWS_PER_W + PAD - 1,), jnp.int32),
        pltpu.VMEM((NCHUNK, CHUNK), jnp.int32),
        pltpu.VMEM((CHUNK, D_MODEL), jnp.float32),
        pltpu.VMEM((CHUNK, D_MODEL), jnp.float32),
        pltpu.VMEM((ROWS_PER_W,), jnp.float32),
        pltpu.SemaphoreType.DMA,
        pltpu.SemaphoreType.DMA,
        pltpu.SemaphoreType.DMA,
        pltpu.SemaphoreType.DMA,
    ],
)
def _emb_lookup(label_hbm, table_hbm, out_hbm, mask_hbm,
                lbl_v, ids_v, buf0, buf1, ones_v, sg0, sg1, sw0, sw1):
    _emb_body(label_hbm, table_hbm, out_hbm, mask_hbm,
              lbl_v, ids_v, buf0, buf1, ones_v, sg0, sg1, sw0, sw1)


def kernel(encoder_hidden_states, label, encoder_attention_mask, embedding_table):
    label_padded = jnp.concatenate(
        [jnp.zeros((PAD,), jnp.int32), label.reshape(N_TOK)])
    out, mask = _emb_lookup(label_padded, embedding_table)
    return (encoder_hidden_states, encoder_attention_mask, out, mask)


# speedup vs baseline: 1.4060x; 1.1185x over previous
import functools

import jax
import jax.numpy as jnp
from jax import lax
from jax.experimental import pallas as pl
from jax.experimental.pallas import tpu as pltpu
from jax.experimental.pallas import tpu_sc as plsc

VOCAB = 32128
D_MODEL = 1024
BATCH = 4
SEQ = 2048
N_TOK = BATCH * SEQ
NC, NS = 2, 16
NW = NC * NS
ROWS_PER_W = N_TOK // NW
CHUNK = 32
NCHUNK = ROWS_PER_W // CHUNK
PAD = 9
LANES = 16

DECODER_START_TOKEN_ID = 0
PAD_TOKEN_ID = 0


def _emb_body(label_hbm, table_hbm, out_hbm, mask_hbm,
              lbl_v, ids_v, buf0, buf1, ones_v, sg0, sg1, sw0, sw1):
    wid = lax.axis_index("s") * NC + lax.axis_index("c")
    base = pl.multiple_of(wid * ROWS_PER_W, ROWS_PER_W)
    b = wid // (SEQ // ROWS_PER_W)
    t_base = pl.multiple_of((wid % (SEQ // ROWS_PER_W)) * ROWS_PER_W,
                            ROWS_PER_W)

    pltpu.sync_copy(label_hbm.at[pl.ds(base, ROWS_PER_W + PAD - 1)], lbl_v)

    lane = lax.iota(jnp.int32, LANES)
    ones16 = jnp.full((LANES,), 1.0, jnp.float32)
    for j in range(ROWS_PER_W // LANES):
        n_vec = base + j * LANES + lane
        is_t0 = (n_vec & (SEQ - 1)) == 0
        ids = lbl_v[pl.ds(PAD - 1 + j * LANES, LANES)]
        ids = jnp.where(ids == -100, PAD_TOKEN_ID, ids)
        ids = jnp.where(is_t0, DECODER_START_TOKEN_ID, ids)
        k, o = divmod(j * LANES, CHUNK)
        ids_v[k, pl.ds(o, LANES)] = ids
        ones_v[pl.ds(j * LANES, LANES)] = ones16
    pltpu.sync_copy(ones_v, mask_hbm.at[b, pl.ds(t_base, ROWS_PER_W)])

    bufs, sgs, sws = [buf0, buf1], [sg0, sg1], [sw0, sw1]

    def start_gather(k):
        return pltpu.async_copy(table_hbm.at[ids_v.at[k]], bufs[k % 2], sgs[k % 2])

    writes = [None] * NCHUNK
    g = start_gather(0)
    for k in range(NCHUNK):
        g.wait()
        writes[k] = pltpu.async_copy(
            bufs[k % 2], out_hbm.at[b, pl.ds(t_base + k * CHUNK, CHUNK)],
            sws[k % 2])
        if k + 1 < NCHUNK:
            if k >= 1:
                writes[k - 1].wait()
            g = start_gather(k + 1)
    writes[NCHUNK - 2].wait()
    writes[NCHUNK - 1].wait()


@functools.partial(
    pl.kernel,
    out_type=(jax.ShapeDtypeStruct((BATCH, SEQ, D_MODEL), jnp.float32),
              jax.ShapeDtypeStruct((BATCH, SEQ), jnp.float32)),
    mesh=plsc.VectorSubcoreMesh(core_axis_name="c", subcore_axis_name="s",
                                num_cores=NC, num_subcores=NS),
    scratch_types=[
        pltpu.VMEM((ROWS_PER_W + PAD - 1,), jnp.int32),
        pltpu.VMEM((NCHUNK, CHUNK), jnp.int32),
        pltpu.VMEM((CHUNK, D_MODEL), jnp.float32),
        pltpu.VMEM((CHUNK, D_MODEL), jnp.float32),
        pltpu.VMEM((ROWS_PER_W,), jnp.float32),
        pltpu.SemaphoreType.DMA,
        pltpu.SemaphoreType.DMA,
        pltpu.SemaphoreType.DMA,
        pltpu.SemaphoreType.DMA,
    ],
)
def _emb_lookup(label_hbm, table_hbm, out_hbm, mask_hbm,
                lbl_v, ids_v, buf0, buf1, ones_v, sg0, sg1, sw0, sw1):
    _emb_body(label_hbm, table_hbm, out_hbm, mask_hbm,
              lbl_v, ids_v, buf0, buf1, ones_v, sg0, sg1, sw0, sw1)


def _copy_body(src_ref, dst_ref):
    dst_ref[...] = src_ref[...]


_EHS_BLOCK = 512


def _tc_passthrough(x):
    flat = x.reshape(N_TOK, D_MODEL)
    spec = pl.BlockSpec((_EHS_BLOCK, D_MODEL), lambda i: (i, 0))
    out = pl.pallas_call(
        _copy_body,
        out_shape=jax.ShapeDtypeStruct((N_TOK, D_MODEL), jnp.float32),
        grid=(N_TOK // _EHS_BLOCK,),
        in_specs=[spec],
        out_specs=spec,
    )(flat)
    return out.reshape(BATCH, SEQ, D_MODEL)


def kernel(encoder_hidden_states, label, encoder_attention_mask, embedding_table):
    label_padded = jnp.concatenate(
        [jnp.zeros((PAD,), jnp.int32), label.reshape(N_TOK)])
    out, mask = _emb_lookup(label_padded, embedding_table)
    ehs = _tc_passthrough(encoder_hidden_states)
    return (ehs, encoder_attention_mask, out, mask)


# speedup vs baseline: 1.4061x; 1.0001x over previous
import functools

import jax
import jax.numpy as jnp
from jax import lax
from jax.experimental import pallas as pl
from jax.experimental.pallas import tpu as pltpu
from jax.experimental.pallas import tpu_sc as plsc

VOCAB = 32128
D_MODEL = 1024
BATCH = 4
SEQ = 2048
N_TOK = BATCH * SEQ
NC, NS = 2, 16
NW = NC * NS
ROWS_PER_W = N_TOK // NW
CHUNK = 32
NCHUNK = ROWS_PER_W // CHUNK
PAD = 9
LANES = 16

DECODER_START_TOKEN_ID = 0
PAD_TOKEN_ID = 0


def _emb_body(label_hbm, table_hbm, out_hbm, mask_hbm,
              lbl_v, ids_v, buf0, buf1, ones_v, sg0, sg1, sw0, sw1):
    wid = lax.axis_index("s") * NC + lax.axis_index("c")
    base = pl.multiple_of(wid * ROWS_PER_W, ROWS_PER_W)
    b = wid // (SEQ // ROWS_PER_W)
    t_base = pl.multiple_of((wid % (SEQ // ROWS_PER_W)) * ROWS_PER_W,
                            ROWS_PER_W)

    pltpu.sync_copy(label_hbm.at[pl.ds(base, ROWS_PER_W + PAD - 1)], lbl_v)

    lane = lax.iota(jnp.int32, LANES)
    ones16 = jnp.full((LANES,), 1.0, jnp.float32)

    def idbody(j, c):
        n_vec = base + j * LANES + lane
        is_t0 = (n_vec & (SEQ - 1)) == 0
        ids = lbl_v[pl.ds(PAD - 1 + j * LANES, LANES)]
        ids = jnp.where(ids == -100, PAD_TOKEN_ID, ids)
        ids = jnp.where(is_t0, DECODER_START_TOKEN_ID, ids)
        ids_v[j // (CHUNK // LANES), pl.ds((j % (CHUNK // LANES)) * LANES, LANES)] = ids
        ones_v[pl.ds(j * LANES, LANES)] = ones16
        return c

    lax.fori_loop(0, ROWS_PER_W // LANES, idbody, 0)
    pltpu.sync_copy(ones_v, mask_hbm.at[b, pl.ds(t_base, ROWS_PER_W)])

    def start_gather(k, buf, sg):
        return pltpu.async_copy(table_hbm.at[ids_v.at[k]], buf, sg)

    def wait_gather(k, buf, sg):
        pltpu.make_async_copy(table_hbm.at[ids_v.at[k]], buf, sg).wait()

    def start_write(k, buf, sw):
        pltpu.async_copy(
            buf, out_hbm.at[b, pl.ds(t_base + k * CHUNK, CHUNK)], sw)

    def wait_write(buf, sw):
        pltpu.make_async_copy(
            buf, out_hbm.at[b, pl.ds(t_base, CHUNK)], sw).wait()

    start_gather(0, buf0, sg0)

    def ring(i, c):
        kk = i * 2
        wait_gather(kk, buf0, sg0)
        start_write(kk, buf0, sw0)

        @pl.when(kk > 0)
        def _():
            wait_write(buf1, sw1)
        start_gather(kk + 1, buf1, sg1)

        wait_gather(kk + 1, buf1, sg1)
        start_write(kk + 1, buf1, sw1)

        @pl.when(kk < NCHUNK - 2)
        def _():
            wait_write(buf0, sw0)
            start_gather(kk + 2, buf0, sg0)
        return c

    lax.fori_loop(0, NCHUNK // 2, ring, 0)
    wait_write(buf0, sw0)
    wait_write(buf1, sw1)


@functools.partial(
    pl.kernel,
    out_type=(jax.ShapeDtypeStruct((BATCH, SEQ, D_MODEL), jnp.float32),
              jax.ShapeDtypeStruct((BATCH, SEQ), jnp.float32)),
    mesh=plsc.VectorSubcoreMesh(core_axis_name="c", subcore_axis_name="s",
                                num_cores=NC, num_subcores=NS),
    scratch_types=[
        pltpu.VMEM((ROWS_PER_W + PAD - 1,), jnp.int32),
        pltpu.VMEM((NCHUNK, CHUNK), jnp.int32),
        pltpu.VMEM((CHUNK, D_MODEL), jnp.float32),
        pltpu.VMEM((CHUNK, D_MODEL), jnp.float32),
        pltpu.VMEM((ROWS_PER_W,), jnp.float32),
        pltpu.SemaphoreType.DMA,
        pltpu.SemaphoreType.DMA,
        pltpu.SemaphoreType.DMA,
        pltpu.SemaphoreType.DMA,
    ],
)
def _emb_lookup(label_hbm, table_hbm, out_hbm, mask_hbm,
                lbl_v, ids_v, buf0, buf1, ones_v, sg0, sg1, sw0, sw1):
    _emb_body(label_hbm, table_hbm, out_hbm, mask_hbm,
              lbl_v, ids_v, buf0, buf1, ones_v, sg0, sg1, sw0, sw1)


def _copy_body(src_ref, dst_ref):
    dst_ref[...] = src_ref[...]


_EHS_BLOCK = 512


def _tc_passthrough(x):
    flat = x.reshape(N_TOK, D_MODEL)
    spec = pl.BlockSpec((_EHS_BLOCK, D_MODEL), lambda i: (i, 0))
    out = pl.pallas_call(
        _copy_body,
        out_shape=jax.ShapeDtypeStruct((N_TOK, D_MODEL), jnp.float32),
        grid=(N_TOK // _EHS_BLOCK,),
        in_specs=[spec],
        out_specs=spec,
    )(flat)
    return out.reshape(BATCH, SEQ, D_MODEL)


def kernel(encoder_hidden_states, label, encoder_attention_mask, embedding_table):
    label_padded = jnp.concatenate(
        [jnp.zeros((PAD,), jnp.int32), label.reshape(N_TOK)])
    out, mask = _emb_lookup(label_padded, embedding_table)
    ehs = _tc_passthrough(encoder_hidden_states)
    return (ehs, encoder_attention_mask, out, mask)


# speedup vs baseline: 1.4240x; 1.0127x over previous
import functools

import jax
import jax.numpy as jnp
from jax import lax
from jax.experimental import pallas as pl
from jax.experimental.pallas import tpu as pltpu
from jax.experimental.pallas import tpu_sc as plsc

VOCAB = 32128
D_MODEL = 1024
BATCH = 4
SEQ = 2048
N_TOK = BATCH * SEQ
NC, NS = 2, 16
NW = NC * NS
ROWS_PER_W = N_TOK // NW
CHUNK = 32
NCHUNK = ROWS_PER_W // CHUNK
PAD = 9
LANES = 16

DECODER_START_TOKEN_ID = 0
PAD_TOKEN_ID = 0


def _emb_body(label_hbm, table_hbm, out_hbm, mask_hbm,
              lbl_v, ids_v, buf0, buf1, ones_v, sg0, sg1, sw0, sw1):
    wid = lax.axis_index("s") * NC + lax.axis_index("c")
    base = pl.multiple_of(wid * ROWS_PER_W, ROWS_PER_W)
    b = wid // (SEQ // ROWS_PER_W)
    t_base = pl.multiple_of((wid % (SEQ // ROWS_PER_W)) * ROWS_PER_W,
                            ROWS_PER_W)

    pltpu.sync_copy(label_hbm.at[pl.ds(base, ROWS_PER_W + PAD - 1)], lbl_v)

    lane = lax.iota(jnp.int32, LANES)
    ones16 = jnp.full((LANES,), 1.0, jnp.float32)

    def idbody(j, c):
        n_vec = base + j * LANES + lane
        is_t0 = (n_vec & (SEQ - 1)) == 0
        ids = lbl_v[pl.ds(PAD - 1 + j * LANES, LANES)]
        ids = jnp.where(ids == -100, PAD_TOKEN_ID, ids)
        ids = jnp.where(is_t0, DECODER_START_TOKEN_ID, ids)
        ids_v[j // (CHUNK // LANES), pl.ds((j % (CHUNK // LANES)) * LANES, LANES)] = ids
        ones_v[pl.ds(j * LANES, LANES)] = ones16
        return c

    lax.fori_loop(0, ROWS_PER_W // LANES, idbody, 0)
    pltpu.sync_copy(ones_v, mask_hbm.at[b, pl.ds(t_base, ROWS_PER_W)])

    def start_gather(k, buf, sg):
        return pltpu.async_copy(table_hbm.at[ids_v.at[k]], buf, sg)

    def wait_gather(k, buf, sg):
        pltpu.make_async_copy(table_hbm.at[ids_v.at[k]], buf, sg).wait()

    def start_write(k, buf, sw):
        pltpu.async_copy(
            buf, out_hbm.at[b, pl.ds(t_base + k * CHUNK, CHUNK)], sw)

    def wait_write(buf, sw):
        pltpu.make_async_copy(
            buf, out_hbm.at[b, pl.ds(t_base, CHUNK)], sw).wait()

    start_gather(0, buf0, sg0)

    def ring(i, c):
        kk = i * 2
        wait_gather(kk, buf0, sg0)
        start_write(kk, buf0, sw0)

        @pl.when(kk > 0)
        def _():
            wait_write(buf1, sw1)
        start_gather(kk + 1, buf1, sg1)

        wait_gather(kk + 1, buf1, sg1)
        start_write(kk + 1, buf1, sw1)

        @pl.when(kk < NCHUNK - 2)
        def _():
            wait_write(buf0, sw0)
            start_gather(kk + 2, buf0, sg0)
        return c

    lax.fori_loop(0, NCHUNK // 2, ring, 0)
    wait_write(buf0, sw0)
    wait_write(buf1, sw1)


@functools.partial(
    pl.kernel,
    out_type=(jax.ShapeDtypeStruct((BATCH, SEQ, D_MODEL), jnp.float32),
              jax.ShapeDtypeStruct((BATCH, SEQ), jnp.float32)),
    mesh=plsc.VectorSubcoreMesh(core_axis_name="c", subcore_axis_name="s",
                                num_cores=NC, num_subcores=NS),
    scratch_types=[
        pltpu.VMEM((ROWS_PER_W + PAD - 1,), jnp.int32),
        pltpu.VMEM((NCHUNK, CHUNK), jnp.int32),
        pltpu.VMEM((CHUNK, D_MODEL), jnp.float32),
        pltpu.VMEM((CHUNK, D_MODEL), jnp.float32),
        pltpu.VMEM((ROWS_PER_W,), jnp.float32),
        pltpu.SemaphoreType.DMA,
        pltpu.SemaphoreType.DMA,
        pltpu.SemaphoreType.DMA,
        pltpu.SemaphoreType.DMA,
    ],
)
def _emb_lookup(label_hbm, table_hbm, out_hbm, mask_hbm,
                lbl_v, ids_v, buf0, buf1, ones_v, sg0, sg1, sw0, sw1):
    _emb_body(label_hbm, table_hbm, out_hbm, mask_hbm,
              lbl_v, ids_v, buf0, buf1, ones_v, sg0, sg1, sw0, sw1)


def _copy_body(src_ref, am_ref, dst_ref, am_out_ref):
    dst_ref[...] = src_ref[...]
    am_out_ref[...] = am_ref[...]


_EHS_BLOCK = 512


def _tc_passthrough(x, attn_mask):
    flat = x.reshape(N_TOK, D_MODEL)
    spec = pl.BlockSpec((_EHS_BLOCK, D_MODEL), lambda i: (i, 0))
    am_spec = pl.BlockSpec((BATCH, SEQ), lambda i: (0, 0))
    out, am = pl.pallas_call(
        _copy_body,
        out_shape=(jax.ShapeDtypeStruct((N_TOK, D_MODEL), jnp.float32),
                   jax.ShapeDtypeStruct((BATCH, SEQ), jnp.float32)),
        grid=(N_TOK // _EHS_BLOCK,),
        in_specs=[spec, am_spec],
        out_specs=(spec, am_spec),
    )(flat, attn_mask)
    return out.reshape(BATCH, SEQ, D_MODEL), am


def kernel(encoder_hidden_states, label, encoder_attention_mask, embedding_table):
    label_padded = jnp.concatenate(
        [jnp.zeros((PAD,), jnp.int32), label.reshape(N_TOK)])
    out, mask = _emb_lookup(label_padded, embedding_table)
    ehs, attn = _tc_passthrough(encoder_hidden_states, encoder_attention_mask)
    return (ehs, attn, out, mask)
